# baseline (speedup 1.0000x reference)
import functools

import jax
import jax.numpy as jnp
from jax import lax
from jax.experimental import pallas as pl
from jax.experimental.pallas import tpu as pltpu
from jax.experimental.pallas import tpu_sc as plsc

_VOCAB = 1000000
_EMBED = 64
_B = 4096
_L = 50
_WIN = 3
_HID = 1024
_NCLS = 1000

_NIDX = _B * _L
_BB = 256
_FLATW = (_L + _WIN - 1) * _EMBED
_PADW = (_L + 2 * (_WIN - 1)) * _EMBED

_NW = 32
_TPW = _NIDX // _NW
_PSTEP = 128
_NSTEP = _TPW // _PSTEP


def _sc_gather(table2, idx):
    mesh = plsc.VectorSubcoreMesh(core_axis_name="c", subcore_axis_name="s")

    @functools.partial(
        pl.kernel,
        out_type=jax.ShapeDtypeStruct((_NIDX, 2 * _EMBED), jnp.float32),
        mesh=mesh,
        scratch_types=[
            pltpu.VMEM((_TPW,), jnp.int32),
            pltpu.VMEM((_PSTEP, 2 * _EMBED), jnp.float32),
            pltpu.VMEM((_PSTEP, 2 * _EMBED), jnp.float32),
            pltpu.SemaphoreType.DMA,
            pltpu.SemaphoreType.DMA,
            pltpu.SemaphoreType.DMA,
            pltpu.SemaphoreType.DMA,
        ],
    )
    def gather_kernel(
        table_hbm, idx_hbm, out_hbm, idx_v, d0, d1, gs0, gs1, ws0, ws1
    ):
        wid = lax.axis_index("s") * 2 + lax.axis_index("c")
        base = wid * _TPW
        pltpu.sync_copy(idx_hbm.at[pl.ds(base, _TPW)], idx_v)

        @pl.loop(0, _NSTEP, step=2)
        def _(s):
            off0 = s * _PSTEP
            off1 = off0 + _PSTEP
            g0 = pltpu.async_copy(
                table_hbm.at[idx_v.at[pl.ds(off0, _PSTEP)]], d0, gs0
            )
            g1 = pltpu.async_copy(
                table_hbm.at[idx_v.at[pl.ds(off1, _PSTEP)]], d1, gs1
            )
            g0.wait()
            w0 = pltpu.async_copy(
                d0, out_hbm.at[pl.ds(base + off0, _PSTEP)], ws0
            )
            g1.wait()
            w1 = pltpu.async_copy(
                d1, out_hbm.at[pl.ds(base + off1, _PSTEP)], ws1
            )
            w0.wait()
            w1.wait()

    return gather_kernel(table2, idx)


def _mlp_body(emb_ref, h_ref, r0_ref, w1_ref, b1_ref, w2_ref, b2_ref, out_ref, p_ref):
    r0 = jnp.broadcast_to(r0_ref[...], (_BB, _EMBED))
    p_ref[:, : _EMBED] = r0
    p_ref[:, _EMBED : 2 * _EMBED] = r0
    for j in range(_L):
        hj = h_ref[:, j : j + 1] == 1
        left = emb_ref[:, 2 * j * _EMBED : (2 * j + 1) * _EMBED]
        right = emb_ref[:, (2 * j + 1) * _EMBED : (2 * j + 2) * _EMBED]
        p_ref[:, (j + 2) * _EMBED : (j + 3) * _EMBED] = jnp.where(
            hj, right, left
        )
    p_ref[:, _PADW - 2 * _EMBED : _PADW - _EMBED] = r0
    p_ref[:, _PADW - _EMBED :] = r0
    p = p_ref[...]
    flat = jnp.maximum(
        jnp.maximum(p[:, :_FLATW], p[:, _EMBED : _EMBED + _FLATW]),
        p[:, 2 * _EMBED : 2 * _EMBED + _FLATW],
    )
    h = jnp.dot(
        flat.astype(jnp.bfloat16), w1_ref[...], preferred_element_type=jnp.float32
    ) + b1_ref[...]
    h = jnp.maximum(h, 0.0).astype(jnp.bfloat16)
    out_ref[...] = jnp.dot(
        h, w2_ref[...], preferred_element_type=jnp.float32
    ) + b2_ref[...]


def _tc_mlp(embp, halves, row0, w1, b1, w2, b2):
    grid = (_B // _BB,)
    return pl.pallas_call(
        _mlp_body,
        grid=grid,
        in_specs=[
            pl.BlockSpec((_BB, _L * 2 * _EMBED), lambda i: (i, 0)),
            pl.BlockSpec((_BB, _EMBED), lambda i: (i, 0)),
            pl.BlockSpec((1, _EMBED), lambda i: (0, 0)),
            pl.BlockSpec((_FLATW, _HID), lambda i: (0, 0)),
            pl.BlockSpec((1, _HID), lambda i: (0, 0)),
            pl.BlockSpec((_HID, _NCLS), lambda i: (0, 0)),
            pl.BlockSpec((1, _NCLS), lambda i: (0, 0)),
        ],
        out_specs=pl.BlockSpec((_BB, _NCLS), lambda i: (i, 0)),
        out_shape=jax.ShapeDtypeStruct((_B, _NCLS), jnp.float32),
        scratch_shapes=[pltpu.VMEM((_BB, _PADW), jnp.float32)],
    )(embp, halves, row0, w1, b1, w2, b2)


def kernel(x, table, W1, b1, W2, b2):
    xi = x.astype(jnp.int32)
    pair = (xi >> 1).reshape(_NIDX)
    halves = jnp.pad(xi & 1, ((0, 0), (0, _EMBED - _L)))
    table2 = table.reshape(_VOCAB // 2, 2 * _EMBED)
    embp = _sc_gather(table2, pair)
    embp2d = embp.reshape(_B, _L * 2 * _EMBED)
    row0 = lax.slice(table, (0, 0), (1, _EMBED))
    w1 = W1.astype(jnp.bfloat16)
    w2 = W2.astype(jnp.bfloat16)
    return _tc_mlp(
        embp2d, halves, row0, w1, b1.reshape(1, _HID), w2, b2.reshape(1, _NCLS)
    )


# speedup vs baseline: 1.1045x; 1.1045x over previous
import functools

import jax
import jax.numpy as jnp
from jax import lax
from jax.experimental import pallas as pl
from jax.experimental.pallas import tpu as pltpu
from jax.experimental.pallas import tpu_sc as plsc

_VOCAB = 1000000
_EMBED = 64
_B = 4096
_L = 50
_WIN = 3
_HID = 1024
_NCLS = 1000

_NIDX = _B * _L
_BB = 256
_FLATW = (_L + _WIN - 1) * _EMBED
_PADW = (_L + 2 * (_WIN - 1)) * _EMBED
_GW = 128


def _sc_gather(table2, idx):
    mesh = plsc.VectorSubcoreMesh(core_axis_name="c", subcore_axis_name="s")

    @functools.partial(
        pl.kernel,
        out_type=jax.ShapeDtypeStruct((_NIDX, 2 * _EMBED), jnp.float32),
        mesh=mesh,
    )
    def gather_kernel(table_hbm, idx_hbm, out_hbm):
        def body(i_vmem, o_vmem):
            pltpu.sync_copy(table_hbm.at[i_vmem.at[0]], o_vmem)

        pltpu.emit_pipeline(
            body,
            grid=(_NIDX // _GW,),
            in_specs=[pl.BlockSpec((1, _GW), index_map=lambda i: (0, i))],
            out_specs=[
                pl.BlockSpec((_GW, 2 * _EMBED), index_map=lambda i: (i, 0))
            ],
            core_axis_name=("c", "s"),
            dimension_semantics=(pltpu.PARALLEL,),
        )(idx_hbm, out_hbm)

    return gather_kernel(table2, idx)


def _mlp_body(emb_ref, h_ref, r0_ref, w1_ref, b1_ref, w2_ref, b2_ref, out_ref, p_ref):
    r0 = jnp.broadcast_to(r0_ref[...], (_BB, _EMBED))
    p_ref[:, : _EMBED] = r0
    p_ref[:, _EMBED : 2 * _EMBED] = r0
    for j in range(_L):
        hj = h_ref[:, j : j + 1] == 1
        pair = emb_ref[j * _BB : (j + 1) * _BB, :]
        p_ref[:, (j + 2) * _EMBED : (j + 3) * _EMBED] = jnp.where(
            hj, pair[:, _EMBED:], pair[:, :_EMBED]
        )
    p_ref[:, _PADW - 2 * _EMBED : _PADW - _EMBED] = r0
    p_ref[:, _PADW - _EMBED :] = r0
    p = p_ref[...]
    flat = jnp.maximum(
        jnp.maximum(p[:, :_FLATW], p[:, _EMBED : _EMBED + _FLATW]),
        p[:, 2 * _EMBED : 2 * _EMBED + _FLATW],
    )
    h = jnp.dot(
        flat.astype(jnp.bfloat16), w1_ref[...], preferred_element_type=jnp.float32
    ) + b1_ref[...]
    h = jnp.maximum(h, 0.0).astype(jnp.bfloat16)
    out_ref[...] = jnp.dot(
        h, w2_ref[...], preferred_element_type=jnp.float32
    ) + b2_ref[...]


def _tc_mlp(embp, halves, row0, w1, b1, w2, b2):
    grid = (_B // _BB,)
    return pl.pallas_call(
        _mlp_body,
        grid=grid,
        in_specs=[
            pl.BlockSpec((_L * _BB, 2 * _EMBED), lambda i: (i, 0)),
            pl.BlockSpec((_BB, _EMBED), lambda i: (i, 0)),
            pl.BlockSpec((1, _EMBED), lambda i: (0, 0)),
            pl.BlockSpec((_FLATW, _HID), lambda i: (0, 0)),
            pl.BlockSpec((1, _HID), lambda i: (0, 0)),
            pl.BlockSpec((_HID, _NCLS), lambda i: (0, 0)),
            pl.BlockSpec((1, _NCLS), lambda i: (0, 0)),
        ],
        out_specs=pl.BlockSpec((_BB, _NCLS), lambda i: (i, 0)),
        out_shape=jax.ShapeDtypeStruct((_B, _NCLS), jnp.float32),
        scratch_shapes=[pltpu.VMEM((_BB, _PADW), jnp.float32)],
    )(embp, halves, row0, w1, b1, w2, b2)


def kernel(x, table, W1, b1, W2, b2):
    xi = x.astype(jnp.int32)
    perm = xi.reshape(_B // _BB, _BB, _L).transpose(0, 2, 1).reshape(1, _NIDX)
    pair = perm >> 1
    halves = jnp.pad(xi & 1, ((0, 0), (0, _EMBED - _L)))
    table2 = table.reshape(_VOCAB // 2, 2 * _EMBED)
    embp = _sc_gather(table2, pair)
    row0 = lax.slice(table, (0, 0), (1, _EMBED))
    w1 = W1.astype(jnp.bfloat16)
    w2 = W2.astype(jnp.bfloat16)
    return _tc_mlp(
        embp, halves, row0, w1, b1.reshape(1, _HID), w2, b2.reshape(1, _NCLS)
    )


# speedup vs baseline: 1.1285x; 1.0217x over previous
import functools

import jax
import jax.numpy as jnp
from jax import lax
from jax.experimental import pallas as pl
from jax.experimental.pallas import tpu as pltpu
from jax.experimental.pallas import tpu_sc as plsc

_VOCAB = 1000000
_EMBED = 64
_B = 4096
_L = 50
_WIN = 3
_HID = 1024
_NCLS = 1000

_NIDX = _B * _L
_BB = 256
_FLATW = (_L + _WIN - 1) * _EMBED
_PADW = (_L + 2 * (_WIN - 1)) * _EMBED
_GW = 128


def _sc_gather(table, idx):
    mesh = plsc.VectorSubcoreMesh(core_axis_name="c", subcore_axis_name="s")

    @functools.partial(
        pl.kernel,
        out_type=jax.ShapeDtypeStruct((_NIDX, _EMBED), jnp.float32),
        mesh=mesh,
        compiler_params=pltpu.CompilerParams(use_tc_tiling_on_sc=False),
    )
    def gather_kernel(table_hbm, idx_hbm, out_hbm):
        def body(i_vmem, o_vmem):
            pltpu.sync_copy(table_hbm.at[i_vmem.at[0]], o_vmem)

        pltpu.emit_pipeline(
            body,
            grid=(_NIDX // _GW,),
            in_specs=[pl.BlockSpec((1, _GW), index_map=lambda i: (0, i))],
            out_specs=[pl.BlockSpec((_GW, _EMBED), index_map=lambda i: (i, 0))],
            core_axis_name=("c", "s"),
            dimension_semantics=(pltpu.PARALLEL,),
        )(idx_hbm, out_hbm)

    return gather_kernel(table, idx)


def _mlp_body(emb_ref, r0_ref, w1_ref, b1_ref, w2_ref, b2_ref, out_ref, p_ref):
    r0 = jnp.broadcast_to(r0_ref[...], (_BB, _EMBED))
    p_ref[:, : _EMBED] = r0
    p_ref[:, _EMBED : 2 * _EMBED] = r0
    for j in range(_L):
        p_ref[:, (j + 2) * _EMBED : (j + 3) * _EMBED] = emb_ref[
            j * _BB : (j + 1) * _BB, :
        ]
    p_ref[:, _PADW - 2 * _EMBED : _PADW - _EMBED] = r0
    p_ref[:, _PADW - _EMBED :] = r0
    p = p_ref[...]
    flat = jnp.maximum(
        jnp.maximum(p[:, :_FLATW], p[:, _EMBED : _EMBED + _FLATW]),
        p[:, 2 * _EMBED : 2 * _EMBED + _FLATW],
    )
    h = jnp.dot(
        flat.astype(jnp.bfloat16), w1_ref[...], preferred_element_type=jnp.float32
    ) + b1_ref[...]
    h = jnp.maximum(h, 0.0).astype(jnp.bfloat16)
    out_ref[...] = jnp.dot(
        h, w2_ref[...], preferred_element_type=jnp.float32
    ) + b2_ref[...]


def _tc_mlp(emb, row0, w1, b1, w2, b2):
    grid = (_B // _BB,)
    return pl.pallas_call(
        _mlp_body,
        grid=grid,
        in_specs=[
            pl.BlockSpec((_L * _BB, _EMBED), lambda i: (i, 0)),
            pl.BlockSpec((1, _EMBED), lambda i: (0, 0)),
            pl.BlockSpec((_FLATW, _HID), lambda i: (0, 0)),
            pl.BlockSpec((1, _HID), lambda i: (0, 0)),
            pl.BlockSpec((_HID, _NCLS), lambda i: (0, 0)),
            pl.BlockSpec((1, _NCLS), lambda i: (0, 0)),
        ],
        out_specs=pl.BlockSpec((_BB, _NCLS), lambda i: (i, 0)),
        out_shape=jax.ShapeDtypeStruct((_B, _NCLS), jnp.float32),
        scratch_shapes=[pltpu.VMEM((_BB, _PADW), jnp.float32)],
    )(emb, row0, w1, b1, w2, b2)


def kernel(x, table, W1, b1, W2, b2):
    xi = x.astype(jnp.int32)
    perm = xi.reshape(_B // _BB, _BB, _L).transpose(0, 2, 1).reshape(1, _NIDX)
    emb = _sc_gather(table, perm)
    row0 = lax.slice(table, (0, 0), (1, _EMBED))
    w1 = W1.astype(jnp.bfloat16)
    w2 = W2.astype(jnp.bfloat16)
    return _tc_mlp(
        emb, row0, w1, b1.reshape(1, _HID), w2, b2.reshape(1, _NCLS)
    )
